# baseline (speedup 1.0000x reference)
import functools

import jax
import jax.numpy as jnp
from jax import lax
from jax.experimental import pallas as pl
from jax.experimental.pallas import tpu as pltpu
from jax.experimental.pallas import tpu_sc as plsc

_BLOCK = 4096
_SC_TOKENS = 4096
_SC_CHUNK = 32


def _tc_body(x_ref, h_ref, out_ref):
    p = h_ref.shape[1]
    proj = jax.lax.dot_general(
        h_ref[...], x_ref[...], (((0,), (1,)), ((), ())),
        preferred_element_type=jnp.float32)
    m = jnp.max(proj, axis=0, keepdims=True)
    iota = jax.lax.broadcasted_iota(jnp.int32, (p, 1), 0)
    w = (jnp.int32(1) << (p - 1 - iota)).astype(jnp.float32)
    eq = (proj == m).astype(jnp.float32)
    v = jnp.sum(eq * w, axis=0)
    e = (jax.lax.bitcast_convert_type(v, jnp.int32) >> 23) - 127
    out_ref[...] = (p - 1 - e).astype(jnp.int32)


def _tc_router(x, hyperplanes, nt):
    t, d = x.shape
    p = hyperplanes.shape[1]
    b = _BLOCK
    return pl.pallas_call(
        _tc_body,
        grid=(nt // b,),
        in_specs=[
            pl.BlockSpec((b, d), lambda i: (i, 0)),
            pl.BlockSpec((d, p), lambda i: (0, 0)),
        ],
        out_specs=pl.BlockSpec((b,), lambda i: (i,)),
        out_shape=jax.ShapeDtypeStruct((nt,), jnp.int32),
    )(x, hyperplanes)


def _sc_router(x, hyperplanes, start, s):
    t, d = x.shape
    p = hyperplanes.shape[1]
    info = plsc.get_sparse_core_info()
    nc, ns = info.num_cores, info.num_subcores
    nw = nc * ns
    tpw = s // nw
    chunk = _SC_CHUNK
    mesh = plsc.VectorSubcoreMesh(core_axis_name="c", subcore_axis_name="s")

    @functools.partial(
        pl.kernel, mesh=mesh,
        out_type=jax.ShapeDtypeStruct((s,), jnp.int32),
        compiler_params=pltpu.CompilerParams(needs_layout_passes=False),
        scratch_types=[
            pltpu.VMEM((d * p,), jnp.float32),
            pltpu.VMEM((chunk, d), jnp.float32),
            pltpu.VMEM((chunk,), jnp.int32),
        ],
    )
    def k(x_hbm, h_hbm, out_hbm, h_v, xbuf, idxbuf):
        wid = lax.axis_index("s") * nc + lax.axis_index("c")
        base = start + wid * tpw
        obase = wid * tpw
        pltpu.sync_copy(h_hbm, h_v)
        lane = lax.broadcasted_iota(jnp.int32, (p,), 0)

        def chunk_body(ci, _):
            cbase = ci * chunk
            pltpu.sync_copy(x_hbm.at[pl.ds(base + cbase, chunk)], xbuf)

            def grp_body(g, _):
                def tok_body(j, idxv):
                    tt = g * p + j

                    def dbody(db, acc):
                        xv = xbuf[tt, pl.ds(db * 16, 16)]
                        u = lax.bitcast_convert_type(xv, jnp.int32)
                        r = (u + 0x7FFF + ((u >> 16) & 1)) & jnp.int32(-65536)
                        xr = lax.bitcast_convert_type(r, jnp.float32)
                        for ll in range(16):
                            acc = acc + xr[ll] * h_v[pl.ds((db * 16 + ll) * p, p)]
                        return acc

                    acc = lax.fori_loop(0, d // 16, dbody,
                                        jnp.zeros((p,), jnp.float32),
                                        unroll=2)
                    m = jnp.max(acc)
                    ffs = plsc.all_reduce_ffs(acc == m)
                    return jnp.where(lane == j, ffs, idxv)

                idxv = lax.fori_loop(0, p, tok_body,
                                     jnp.zeros((p,), jnp.int32))
                idxbuf[pl.ds(g * p, p)] = idxv
                return 0

            lax.fori_loop(0, chunk // p, grp_body, 0)
            pltpu.sync_copy(idxbuf, out_hbm.at[pl.ds(obase + cbase, chunk)])
            return 0

        lax.fori_loop(0, tpw // chunk, chunk_body, 0)

    u = lax.bitcast_convert_type(hyperplanes, jnp.int32)
    r = (u + 0x7FFF + ((u >> 16) & 1)) & jnp.int32(-65536)
    h_rounded = lax.bitcast_convert_type(r, jnp.float32)
    return k(x, h_rounded.reshape(-1))


def kernel(x, hyperplanes):
    t, _ = x.shape
    s = _SC_TOKENS
    nt = t - s
    out_sc = _sc_router(x, hyperplanes, nt, s)
    out_tc = _tc_router(x, hyperplanes, nt)
    return jnp.concatenate([out_tc, out_sc])


# speedup vs baseline: 2.3710x; 2.3710x over previous
import functools

import jax
import jax.numpy as jnp
from jax import lax
from jax.experimental import pallas as pl
from jax.experimental.pallas import tpu as pltpu
from jax.experimental.pallas import tpu_sc as plsc

_BLOCK = 4096
_SC_TOKENS = 2048
_SC_CHUNK = 32


def _tc_body(x_ref, h_ref, out_ref):
    p = h_ref.shape[1]
    proj = jax.lax.dot_general(
        h_ref[...], x_ref[...], (((0,), (1,)), ((), ())),
        preferred_element_type=jnp.float32)
    m = jnp.max(proj, axis=0, keepdims=True)
    iota = jax.lax.broadcasted_iota(jnp.int32, (p, 1), 0)
    w = (jnp.int32(1) << (p - 1 - iota)).astype(jnp.float32)
    eq = (proj == m).astype(jnp.float32)
    v = jnp.sum(eq * w, axis=0)
    e = (jax.lax.bitcast_convert_type(v, jnp.int32) >> 23) - 127
    out_ref[...] = (p - 1 - e).astype(jnp.int32)


def _tc_router(x, hyperplanes, nt):
    t, d = x.shape
    p = hyperplanes.shape[1]
    b = _BLOCK if nt % _BLOCK == 0 else 2048
    return pl.pallas_call(
        _tc_body,
        grid=(nt // b,),
        in_specs=[
            pl.BlockSpec((b, d), lambda i: (i, 0)),
            pl.BlockSpec((d, p), lambda i: (0, 0)),
        ],
        out_specs=pl.BlockSpec((b,), lambda i: (i,)),
        out_shape=jax.ShapeDtypeStruct((nt,), jnp.int32),
    )(x, hyperplanes)


def _sc_router(x, hyperplanes, start, s):
    t, d = x.shape
    p = hyperplanes.shape[1]
    info = plsc.get_sparse_core_info()
    nc, ns = info.num_cores, info.num_subcores
    nw = nc * ns
    tpw = s // nw
    chunk = _SC_CHUNK
    mesh = plsc.VectorSubcoreMesh(core_axis_name="c", subcore_axis_name="s")

    @functools.partial(
        pl.kernel, mesh=mesh,
        out_type=jax.ShapeDtypeStruct((s,), jnp.int32),
        compiler_params=pltpu.CompilerParams(needs_layout_passes=False),
        scratch_types=[
            pltpu.VMEM((d * p,), jnp.float32),
            pltpu.VMEM((chunk, d), jnp.float32),
            pltpu.VMEM((chunk,), jnp.int32),
        ],
    )
    def k(x_hbm, h_hbm, out_hbm, h_v, xbuf, idxbuf):
        wid = lax.axis_index("s") * nc + lax.axis_index("c")
        base = start + wid * tpw
        obase = wid * tpw
        pltpu.sync_copy(h_hbm, h_v)
        lane = lax.broadcasted_iota(jnp.int32, (p,), 0)

        def chunk_body(ci, _):
            cbase = ci * chunk
            pltpu.sync_copy(x_hbm.at[pl.ds(base + cbase, chunk)], xbuf)

            def grp_body(g, _):
                def tok_body(j, idxv):
                    tt = g * p + j

                    def dbody(db, accs):
                        xv = xbuf[tt, pl.ds(db * 16, 16)]
                        u = lax.bitcast_convert_type(xv, jnp.int32)
                        r = (u + 0x7FFF + ((u >> 16) & 1)) & jnp.int32(-65536)
                        xr = lax.bitcast_convert_type(r, jnp.float32)
                        accs = list(accs)
                        for ll in range(16):
                            accs[ll % 4] = (
                                accs[ll % 4]
                                + xr[ll] * h_v[pl.ds((db * 16 + ll) * p, p)])
                        return tuple(accs)

                    z = jnp.zeros((p,), jnp.float32)
                    a0, a1, a2, a3 = lax.fori_loop(0, d // 16, dbody,
                                                   (z, z, z, z), unroll=2)
                    acc = (a0 + a1) + (a2 + a3)
                    m = jnp.max(acc)
                    ffs = plsc.all_reduce_ffs(acc == m)
                    return jnp.where(lane == j, ffs, idxv)

                idxv = lax.fori_loop(0, p, tok_body,
                                     jnp.zeros((p,), jnp.int32))
                idxbuf[pl.ds(g * p, p)] = idxv
                return 0

            lax.fori_loop(0, chunk // p, grp_body, 0)
            pltpu.sync_copy(idxbuf, out_hbm.at[pl.ds(obase + cbase, chunk)])
            return 0

        lax.fori_loop(0, tpw // chunk, chunk_body, 0)

    u = lax.bitcast_convert_type(hyperplanes, jnp.int32)
    r = (u + 0x7FFF + ((u >> 16) & 1)) & jnp.int32(-65536)
    h_rounded = lax.bitcast_convert_type(r, jnp.float32)
    return k(x, h_rounded.reshape(-1))


def kernel(x, hyperplanes):
    t, _ = x.shape
    s = _SC_TOKENS
    nt = t - s
    out_sc = _sc_router(x, hyperplanes, nt, s)
    out_tc = _tc_router(x, hyperplanes, nt)
    return jnp.concatenate([out_tc, out_sc])


# speedup vs baseline: 2.6618x; 1.1227x over previous
import functools

import jax
import jax.numpy as jnp
from jax import lax
from jax.experimental import pallas as pl
from jax.experimental.pallas import tpu as pltpu
from jax.experimental.pallas import tpu_sc as plsc

_BLOCK = 4096
_SC_TOKENS = 1024
_SC_CHUNK = 32


def _tc_body(x_ref, h_ref, out_ref):
    p = h_ref.shape[1]
    proj = jax.lax.dot_general(
        h_ref[...], x_ref[...], (((0,), (1,)), ((), ())),
        preferred_element_type=jnp.float32)
    m = jnp.max(proj, axis=0, keepdims=True)
    iota = jax.lax.broadcasted_iota(jnp.int32, (p, 1), 0)
    w = (jnp.int32(1) << (p - 1 - iota)).astype(jnp.float32)
    eq = (proj == m).astype(jnp.float32)
    v = jnp.sum(eq * w, axis=0)
    e = (jax.lax.bitcast_convert_type(v, jnp.int32) >> 23) - 127
    out_ref[...] = (p - 1 - e).astype(jnp.int32)


def _tc_router(x, hyperplanes, nt):
    t, d = x.shape
    p = hyperplanes.shape[1]
    b = _BLOCK if nt % _BLOCK == 0 else 2048
    return pl.pallas_call(
        _tc_body,
        grid=(nt // b,),
        in_specs=[
            pl.BlockSpec((b, d), lambda i: (i, 0)),
            pl.BlockSpec((d, p), lambda i: (0, 0)),
        ],
        out_specs=pl.BlockSpec((b,), lambda i: (i,)),
        out_shape=jax.ShapeDtypeStruct((nt,), jnp.int32),
    )(x, hyperplanes)


def _sc_router(x, hyperplanes, start, s):
    t, d = x.shape
    p = hyperplanes.shape[1]
    info = plsc.get_sparse_core_info()
    nc, ns = info.num_cores, info.num_subcores
    nw = nc * ns
    tpw = s // nw
    chunk = _SC_CHUNK
    mesh = plsc.VectorSubcoreMesh(core_axis_name="c", subcore_axis_name="s")

    @functools.partial(
        pl.kernel, mesh=mesh,
        out_type=jax.ShapeDtypeStruct((s,), jnp.int32),
        compiler_params=pltpu.CompilerParams(needs_layout_passes=False),
        scratch_types=[
            pltpu.VMEM((d * p,), jnp.float32),
            pltpu.VMEM((chunk, d), jnp.float32),
            pltpu.VMEM((chunk,), jnp.int32),
        ],
    )
    def k(x_hbm, h_hbm, out_hbm, h_v, xbuf, idxbuf):
        wid = lax.axis_index("s") * nc + lax.axis_index("c")
        base = start + wid * tpw
        obase = wid * tpw
        pltpu.sync_copy(h_hbm, h_v)
        lane = lax.broadcasted_iota(jnp.int32, (p,), 0)

        def chunk_body(ci, _):
            cbase = ci * chunk
            pltpu.sync_copy(x_hbm.at[pl.ds(base + cbase, chunk)], xbuf)

            def grp_body(g, _):
                def tok_body(j, idxv):
                    tt = g * p + j

                    def dbody(db, accs):
                        xv = xbuf[tt, pl.ds(db * 16, 16)]
                        u = lax.bitcast_convert_type(xv, jnp.int32)
                        r = (u + 0x7FFF + ((u >> 16) & 1)) & jnp.int32(-65536)
                        xr = lax.bitcast_convert_type(r, jnp.float32)
                        accs = list(accs)
                        for ll in range(16):
                            accs[ll % 4] = (
                                accs[ll % 4]
                                + xr[ll] * h_v[pl.ds((db * 16 + ll) * p, p)])
                        return tuple(accs)

                    z = jnp.zeros((p,), jnp.float32)
                    a0, a1, a2, a3 = lax.fori_loop(0, d // 16, dbody,
                                                   (z, z, z, z), unroll=2)
                    acc = (a0 + a1) + (a2 + a3)
                    m = jnp.max(acc)
                    ffs = plsc.all_reduce_ffs(acc == m)
                    return jnp.where(lane == j, ffs, idxv)

                idxv = lax.fori_loop(0, p, tok_body,
                                     jnp.zeros((p,), jnp.int32))
                idxbuf[pl.ds(g * p, p)] = idxv
                return 0

            lax.fori_loop(0, chunk // p, grp_body, 0)
            pltpu.sync_copy(idxbuf, out_hbm.at[pl.ds(obase + cbase, chunk)])
            return 0

        lax.fori_loop(0, tpw // chunk, chunk_body, 0)

    u = lax.bitcast_convert_type(hyperplanes, jnp.int32)
    r = (u + 0x7FFF + ((u >> 16) & 1)) & jnp.int32(-65536)
    h_rounded = lax.bitcast_convert_type(r, jnp.float32)
    return k(x, h_rounded.reshape(-1))


def kernel(x, hyperplanes):
    t, _ = x.shape
    s = _SC_TOKENS
    nt = t - s
    out_sc = _sc_router(x, hyperplanes, nt, s)
    out_tc = _tc_router(x, hyperplanes, nt)
    return jnp.concatenate([out_tc, out_sc])


# speedup vs baseline: 4.2622x; 1.6012x over previous
import jax
import jax.numpy as jnp
from jax.experimental import pallas as pl

_BLOCK = 4096


def _body(x_ref, h_ref, out_ref):
    p = h_ref.shape[1]
    proj = jax.lax.dot_general(
        h_ref[...], x_ref[...], (((0,), (1,)), ((), ())),
        preferred_element_type=jnp.float32)
    m = jnp.max(proj, axis=0, keepdims=True)
    iota = jax.lax.broadcasted_iota(jnp.int32, (p, 1), 0)
    w = (jnp.int32(1) << (p - 1 - iota)).astype(jnp.float32)
    eq = (proj == m).astype(jnp.float32)
    v = jnp.sum(eq * w, axis=0)
    e = (jax.lax.bitcast_convert_type(v, jnp.int32) >> 23) - 127
    out_ref[...] = (p - 1 - e).astype(jnp.int32)


def kernel(x, hyperplanes):
    t, d = x.shape
    p = hyperplanes.shape[1]
    b = _BLOCK
    return pl.pallas_call(
        _body,
        grid=(t // b,),
        in_specs=[
            pl.BlockSpec((b, d), lambda i: (i, 0)),
            pl.BlockSpec((d, p), lambda i: (0, 0)),
        ],
        out_specs=pl.BlockSpec((b,), lambda i: (i,)),
        out_shape=jax.ShapeDtypeStruct((t,), jnp.int32),
    )(x, hyperplanes)
